# baseline (speedup 1.0000x reference)
import functools

import jax
import jax.numpy as jnp
from jax import lax
from jax.experimental import pallas as pl
from jax.experimental.pallas import tpu as pltpu
from jax.experimental.pallas import tpu_sc as plsc

_VOCAB = 1000
_EMBED = 64
_NW = 32
_CH = 128
_LANES = 16


def _make_sc_gather(n_tokens):
    chunk = n_tokens // _NW
    nch = chunk // _CH
    nvec = chunk // _LANES
    mesh = plsc.VectorSubcoreMesh(core_axis_name="c", subcore_axis_name="s")

    @functools.partial(
        pl.kernel,
        mesh=mesh,
        compiler_params=pltpu.CompilerParams(use_tc_tiling_on_sc=False),
        out_type=jax.ShapeDtypeStruct((n_tokens, _EMBED), jnp.float32),
        scratch_types=[
            pltpu.VMEM((16 + chunk,), jnp.int32),
            pltpu.VMEM((chunk,), jnp.int32),
            pltpu.VMEM((_CH, _EMBED), jnp.float32),
            pltpu.SemaphoreType.DMA,
        ],
    )
    def sc_gather(tok_hbm, table_hbm, out_hbm, tok_v, keys_v, rows_v, gsem):
        wid = lax.axis_index("s") * 2 + lax.axis_index("c")
        base = wid * chunk
        pltpu.sync_copy(tok_hbm.at[pl.ds(base, chunk)], tok_v.at[pl.ds(16, chunk)])

        @pl.when(wid > 0)
        def _():
            pltpu.sync_copy(tok_hbm.at[pl.ds(base - 8, 8)], tok_v.at[pl.ds(8, 8)])

        def key_body(i, carry):
            cur = tok_v[pl.ds(16 + i * _LANES, _LANES)]
            prv = tok_v[pl.ds(15 + i * _LANES, _LANES)]
            pos = i * _LANES + lax.iota(jnp.int32, _LANES)
            is_row_start = (pos % 200) == 0
            keys_v[pl.ds(i * _LANES, _LANES)] = jnp.where(
                is_row_start, cur, prv * _VOCAB + cur)
            return carry

        lax.fori_loop(0, nvec, key_body, 0)

        def gather_body(k, carry):
            idx = keys_v.at[pl.ds(k * _CH, _CH)]
            pltpu.async_copy(table_hbm.at[idx], rows_v, gsem).wait()
            pltpu.sync_copy(rows_v, out_hbm.at[pl.ds(base + k * _CH, _CH)])
            return carry

        lax.fori_loop(0, nch, gather_body, 0)

    return sc_gather


def _dense_body(h_ref, m_ref, gw_ref, gb_ref, ow_ref, ob_ref, out_ref, gate_ref):
    h = h_ref[...]
    g = jax.nn.sigmoid(
        jnp.dot(h, gw_ref[...], preferred_element_type=jnp.float32) + gb_ref[0, 0])
    y = jnp.dot(m_ref[...], ow_ref[...], preferred_element_type=jnp.float32) + ob_ref[...]
    out_ref[...] = h + g * y
    gate_ref[...] = g


def kernel(x_tokens, hidden_state, memory_table, gate_w, gate_b, out_w, out_b):
    b, l = x_tokens.shape
    n = b * l
    x_flat = x_tokens.reshape(n).astype(jnp.int32)
    gathered = _make_sc_gather(n)(x_flat, memory_table)

    h2 = hidden_state.reshape(n, _EMBED)
    blk = 2048
    grid = (n // blk,)
    out2, gate2 = pl.pallas_call(
        _dense_body,
        grid=grid,
        in_specs=[
            pl.BlockSpec((blk, _EMBED), lambda i: (i, 0)),
            pl.BlockSpec((blk, _EMBED), lambda i: (i, 0)),
            pl.BlockSpec((_EMBED, 1), lambda i: (0, 0)),
            pl.BlockSpec((1, 1), lambda i: (0, 0)),
            pl.BlockSpec((_EMBED, _EMBED), lambda i: (0, 0)),
            pl.BlockSpec((1, _EMBED), lambda i: (0, 0)),
        ],
        out_specs=[
            pl.BlockSpec((blk, _EMBED), lambda i: (i, 0)),
            pl.BlockSpec((blk, 1), lambda i: (i, 0)),
        ],
        out_shape=[
            jax.ShapeDtypeStruct((n, _EMBED), jnp.float32),
            jax.ShapeDtypeStruct((n, 1), jnp.float32),
        ],
    )(h2, gathered, gate_w, gate_b.reshape(1, 1), out_w, out_b.reshape(1, _EMBED))

    return out2.reshape(b, l, _EMBED), gate2.reshape(b, l, 1)


# speedup vs baseline: 1.1221x; 1.1221x over previous
import functools

import jax
import jax.numpy as jnp
from jax import lax
from jax.experimental import pallas as pl
from jax.experimental.pallas import tpu as pltpu
from jax.experimental.pallas import tpu_sc as plsc

_VOCAB = 1000
_EMBED = 64
_NW = 32
_CH = 128
_LANES = 16


def _make_sc_gather(n_tokens):
    chunk = n_tokens // _NW
    nch = chunk // _CH
    nvec = chunk // _LANES
    mesh = plsc.VectorSubcoreMesh(core_axis_name="c", subcore_axis_name="s")

    @functools.partial(
        pl.kernel,
        mesh=mesh,
        compiler_params=pltpu.CompilerParams(use_tc_tiling_on_sc=False),
        out_type=jax.ShapeDtypeStruct((n_tokens, 2 * _EMBED), jnp.float32),
        scratch_types=[
            pltpu.VMEM((16 + chunk,), jnp.int32),
            pltpu.VMEM((nch, _CH), jnp.int32),
            pltpu.VMEM((2, _CH, _EMBED), jnp.float32),
            pltpu.SemaphoreType.DMA,
            pltpu.SemaphoreType.DMA,
        ],
    )
    def sc_gather(tok_hbm, table_hbm, out_hbm, tok_v, keys_v, rows_v, sem0, sem1):
        wid = lax.axis_index("s") * 2 + lax.axis_index("c")
        base = wid * chunk
        pltpu.sync_copy(tok_hbm.at[pl.ds(base, chunk)], tok_v.at[pl.ds(16, chunk)])

        @pl.when(wid > 0)
        def _():
            pltpu.sync_copy(tok_hbm.at[pl.ds(base - 8, 8)], tok_v.at[pl.ds(8, 8)])

        def key_body(i, carry):
            cur = tok_v[pl.ds(16 + i * _LANES, _LANES)]
            prv = tok_v[pl.ds(15 + i * _LANES, _LANES)]
            pos = i * _LANES + lax.iota(jnp.int32, _LANES)
            is_row_start = (pos % 200) == 0
            keys_v[i // 8, pl.ds((i % 8) * _LANES, _LANES)] = jnp.where(
                is_row_start, cur, prv * _VOCAB + cur)
            return carry

        lax.fori_loop(0, nvec, key_body, 0)

        def fire(k, slot, sem):
            return pltpu.async_copy(table_hbm.at[keys_v.at[k]], rows_v.at[slot], sem)

        def drain(k, slot, sem):
            pltpu.make_async_copy(
                table_hbm.at[keys_v.at[k]], rows_v.at[slot], sem).wait()

        def scat(k, slot):
            pltpu.sync_copy(
                rows_v.at[slot],
                out_hbm.at[pl.ds(base + k * _CH, _CH), pl.ds(0, _EMBED)])

        fire(0, 0, sem0)

        def gather_body(k2, carry):
            k0 = k2 * 2
            fire(k0 + 1, 1, sem1)
            drain(k0, 0, sem0)
            scat(k0, 0)

            @pl.when(k0 + 2 < nch)
            def _():
                fire(k0 + 2, 0, sem0)

            drain(k0 + 1, 1, sem1)
            scat(k0 + 1, 1)
            return carry

        lax.fori_loop(0, nch // 2, gather_body, 0)

    return sc_gather


def _dense_body(h_ref, m_ref, gw_ref, gb_ref, ow_ref, ob_ref, out_ref, gate_ref):
    h = h_ref[...]
    m = m_ref[...][:, :_EMBED]
    g = jax.nn.sigmoid(
        jnp.dot(h, gw_ref[...], preferred_element_type=jnp.float32) + gb_ref[0, 0])
    y = jnp.dot(m, ow_ref[...], preferred_element_type=jnp.float32) + ob_ref[...]
    out_ref[...] = h + g * y
    gate_ref[...] = g


def kernel(x_tokens, hidden_state, memory_table, gate_w, gate_b, out_w, out_b):
    b, l = x_tokens.shape
    n = b * l
    x_flat = x_tokens.reshape(n).astype(jnp.int32)
    gathered = _make_sc_gather(n)(x_flat, memory_table)

    h2 = hidden_state.reshape(n, _EMBED)
    blk = 2048
    grid = (n // blk,)
    out2, gate2 = pl.pallas_call(
        _dense_body,
        grid=grid,
        in_specs=[
            pl.BlockSpec((blk, _EMBED), lambda i: (i, 0)),
            pl.BlockSpec((blk, 2 * _EMBED), lambda i: (i, 0)),
            pl.BlockSpec((_EMBED, 1), lambda i: (0, 0)),
            pl.BlockSpec((1, 1), lambda i: (0, 0)),
            pl.BlockSpec((_EMBED, _EMBED), lambda i: (0, 0)),
            pl.BlockSpec((1, _EMBED), lambda i: (0, 0)),
        ],
        out_specs=[
            pl.BlockSpec((blk, _EMBED), lambda i: (i, 0)),
            pl.BlockSpec((blk, 1), lambda i: (i, 0)),
        ],
        out_shape=[
            jax.ShapeDtypeStruct((n, _EMBED), jnp.float32),
            jax.ShapeDtypeStruct((n, 1), jnp.float32),
        ],
    )(h2, gathered, gate_w, gate_b.reshape(1, 1), out_w, out_b.reshape(1, _EMBED))

    return out2.reshape(b, l, _EMBED), gate2.reshape(b, l, 1)
